# baseline (speedup 1.0000x reference)
import functools
import jax
import jax.numpy as jnp
from jax import lax
from jax.experimental import pallas as pl
from jax.experimental.pallas import tpu as pltpu
from jax.experimental.pallas import tpu_sc as plsc

D_M = 64
SCALE = 8.0
NW = 32
LANES = 16
V = 1000000

BLK = 160
NBLK = V // BLK

NBUF = 4
AHEAD = 2
SLAB = 32


def _wid():
    return lax.axis_index("s") * 2 + lax.axis_index("c")


def _padscale_call():
    mesh = plsc.VectorSubcoreMesh(core_axis_name="c", subcore_axis_name="s")
    nper = (NBLK + NW - 1) // NW

    @functools.partial(
        pl.kernel,
        mesh=mesh,
        out_type=jax.ShapeDtypeStruct((V, 2 * D_M), jnp.float32),
        scratch_types=[
            pltpu.VMEM((2, BLK, D_M), jnp.float32),
            pltpu.VMEM((2, BLK, 2 * D_M), jnp.float32),
            pltpu.SemaphoreType.DMA((2,)),
            pltpu.SemaphoreType.DMA((2,)),
        ],
    )
    def body(lut_hbm, lutp_hbm, ibufs, obufs, isems, osems):
        w = _wid()

        def blk_of(t):
            return w + t * NW

        for b in range(2):
            @pl.when(blk_of(b) < NBLK)
            def _p():
                pltpu.async_copy(
                    lut_hbm.at[pl.ds(blk_of(b) * BLK, BLK)], ibufs.at[b],
                    isems.at[b],
                )

        def step(t, carry):
            b = t % 2
            blk = blk_of(t)

            @pl.when(blk < NBLK)
            def _work():
                pltpu.make_async_copy(
                    lut_hbm.at[pl.ds(blk * BLK, BLK)], ibufs.at[b], isems.at[b]
                ).wait()

                def srow(r, c2):
                    for q in range(D_M // LANES):
                        sl = pl.ds(q * LANES, LANES)
                        obufs[b, r, sl] = ibufs[b, r, sl] * SCALE
                    return c2

                lax.fori_loop(0, BLK, srow, 0, unroll=4)

                @pl.when(t >= 2)
                def _drain():
                    pltpu.make_async_copy(
                        obufs.at[b],
                        lutp_hbm.at[pl.ds(blk_of(t - 2) * BLK, BLK)],
                        osems.at[b],
                    ).wait()

                pltpu.async_copy(
                    obufs.at[b], lutp_hbm.at[pl.ds(blk * BLK, BLK)], osems.at[b]
                )

                @pl.when(blk_of(t + 2) < NBLK)
                def _next():
                    pltpu.async_copy(
                        lut_hbm.at[pl.ds(blk_of(t + 2) * BLK, BLK)],
                        ibufs.at[b],
                        isems.at[b],
                    )

            return carry

        lax.fori_loop(0, nper, step, 0)

        for t in range(nper - 2, nper):
            @pl.when(blk_of(t) < NBLK)
            def _d():
                pltpu.make_async_copy(
                    obufs.at[t % 2],
                    lutp_hbm.at[pl.ds(blk_of(t) * BLK, BLK)],
                    osems.at[t % 2],
                ).wait()

    return body


CA = 128
CB = 72


def _gather_call(R, C):
    RW = R // NW
    NS = RW // SLAB
    NCH = 2 * RW
    mesh = plsc.VectorSubcoreMesh(core_axis_name="c", subcore_axis_name="s")

    @functools.partial(
        pl.kernel,
        mesh=mesh,
        out_type=jax.ShapeDtypeStruct((R, C, 2 * D_M), jnp.float32),
        scratch_types=[
            pltpu.VMEM((2, SLAB, CA), jnp.int32),
            pltpu.VMEM((2, SLAB, CB), jnp.int32),
            pltpu.VMEM((2, CA, 2 * D_M), jnp.float32),
            pltpu.VMEM((2, CB, 2 * D_M), jnp.float32),
            pltpu.SemaphoreType.DMA((2,)),
            pltpu.SemaphoreType.DMA((2,)),
            pltpu.SemaphoreType.DMA((2,)),
            pltpu.SemaphoreType.DMA((2,)),
            pltpu.SemaphoreType.DMA((2,)),
            pltpu.SemaphoreType.DMA((2,)),
        ],
    )
    def body(idx_hbm, lutp_hbm, out_hbm, iA, iB, bufA, bufB,
             isemA, isemB, gsemA, gsemB, osemA, osemB):
        rbase = _wid() * RW

        def stage(s, sem_wait=False):
            rows = pl.ds(rbase + s * SLAB, SLAB)
            cpa = (idx_hbm.at[rows, pl.ds(0, CA)], iA.at[s % 2], isemA.at[s % 2])
            cpb = (idx_hbm.at[rows, pl.ds(CA, CB)], iB.at[s % 2], isemB.at[s % 2])
            if sem_wait:
                pltpu.make_async_copy(*cpa).wait()
                pltpu.make_async_copy(*cpb).wait()
            else:
                pltpu.async_copy(*cpa)
                pltpu.async_copy(*cpb)

        def gather(r, part, h, start):
            sb = (r // SLAB) % 2
            rs = r % SLAB
            if part == 0:
                args = (lutp_hbm.at[iA.at[sb, rs]], bufA.at[h], gsemA.at[h])
            else:
                args = (lutp_hbm.at[iB.at[sb, rs]], bufB.at[h], gsemB.at[h])
            if start:
                pltpu.async_copy(*args)
            else:
                pltpu.make_async_copy(*args).wait()

        def store(r, part, h, start):
            if part == 0:
                args = (
                    bufA.at[h],
                    out_hbm.at[rbase + r, pl.ds(0, CA)],
                    osemA.at[h],
                )
            else:
                args = (
                    bufB.at[h],
                    out_hbm.at[rbase + r, pl.ds(CA, CB)],
                    osemB.at[h],
                )
            if start:
                pltpu.async_copy(*args)
            else:
                pltpu.make_async_copy(*args).wait()

        stage(0)
        stage(1)
        stage(0, sem_wait=True)
        gather(0, 0, 0, start=True)
        gather(0, 1, 0, start=True)

        def slab_loop(s, carry):
            @pl.when(s + 1 < NS)
            def _wait_next():
                stage(s + 1, sem_wait=True)

            def block(t, c1):
                for u in range(4):
                    hr = u // 2
                    part = u % 2
                    r = 2 * t + hr
                    hn = (hr + 1) % 2

                    @pl.when(r + 1 < RW)
                    def _fire():
                        @pl.when(r >= 1)
                        def _drain():
                            store(r - 1, part, hn, start=False)

                        gather(r + 1, part, hn, start=True)

                    gather(r, part, hr, start=False)
                    store(r, part, hr, start=True)
                return c1

            lax.fori_loop(s * (SLAB // 2), (s + 1) * (SLAB // 2), block, 0)

            @pl.when(s + 2 < NS)
            def _restage():
                stage(s + 2)

            return carry

        lax.fori_loop(0, NS, slab_loop, 0)

        for r in (RW - 2, RW - 1):
            for part in (0, 1):
                store(r, part, r % 2, start=False)

    return body


def kernel(x, lut):
    xi = x.astype(jnp.int32)
    lutp = _padscale_call()(lut)
    outp = _gather_call(x.shape[0], x.shape[1])(xi, lutp)
    return outp[:, :, :D_M]


# speedup vs baseline: 1.4873x; 1.4873x over previous
import functools
import jax
import jax.numpy as jnp
from jax import lax
from jax.experimental import pallas as pl
from jax.experimental.pallas import tpu as pltpu
from jax.experimental.pallas import tpu_sc as plsc

D_M = 64
PAD_W = 128
SCALE = 8.0
NW = 32
LANES = 16
NBUF = 4
AHEAD = 2
SLAB = 32


def _gather_call(R, C):
    RW = R // NW
    NS = RW // SLAB
    mesh = plsc.VectorSubcoreMesh(core_axis_name="c", subcore_axis_name="s")

    @functools.partial(
        pl.kernel,
        mesh=mesh,
        out_type=jax.ShapeDtypeStruct((R, C, PAD_W), jnp.float32),
        compiler_params=pltpu.CompilerParams(use_tc_tiling_on_sc=False),
        scratch_types=[
            pltpu.VMEM((2, SLAB, C), jnp.int32),
            pltpu.VMEM((NBUF, C, PAD_W), jnp.float32),
            pltpu.SemaphoreType.DMA((2,)),
            pltpu.SemaphoreType.DMA((NBUF,)),
            pltpu.SemaphoreType.DMA((NBUF,)),
        ],
    )
    def body(idx_hbm, lutp_hbm, out_hbm, islabs, bufs, isems, gsems, osems):
        rbase = _wid_expr() * RW

        def stage(s, sem_wait=False):
            args = (
                idx_hbm.at[pl.ds(rbase + s * SLAB, SLAB)],
                islabs.at[s % 2],
                isems.at[s % 2],
            )
            if sem_wait:
                pltpu.make_async_copy(*args).wait()
            else:
                pltpu.async_copy(*args)

        def idx_row(j):
            return islabs.at[(j // SLAB) % 2, j % SLAB]

        stage(0)
        stage(1)
        stage(0, sem_wait=True)
        for b in range(AHEAD):
            pltpu.async_copy(lutp_hbm.at[idx_row(b)], bufs.at[b], gsems.at[b])

        def slab_loop(s, carry):
            @pl.when(s + 1 < NS)
            def _wait_next():
                stage(s + 1, sem_wait=True)

            def block(j0, c1):
                for b in range(NBUF):
                    j = j0 + b
                    jf = j + AHEAD
                    bf = (b + AHEAD) % NBUF

                    @pl.when(jf < RW)
                    def _fire():
                        @pl.when(jf >= NBUF)
                        def _drain():
                            pltpu.make_async_copy(
                                bufs.at[bf, :, pl.ds(0, D_M)],
                                out_hbm.at[rbase + jf - NBUF, :, pl.ds(0, D_M)],
                                osems.at[bf],
                            ).wait()

                        pltpu.async_copy(
                            lutp_hbm.at[idx_row(jf)], bufs.at[bf], gsems.at[bf]
                        )

                    pltpu.make_async_copy(
                        lutp_hbm.at[idx_row(j)], bufs.at[b], gsems.at[b]
                    ).wait()

                    def srow(t, c2):
                        for rr in range(2):
                            for q in range(D_M // LANES):
                                sl = pl.ds(q * LANES, LANES)
                                bufs[b, 2 * t + rr, sl] = (
                                    bufs[b, 2 * t + rr, sl] * SCALE
                                )
                        return c2

                    lax.fori_loop(0, C // 2, srow, 0, unroll=2)

                    pltpu.async_copy(
                        bufs.at[b, :, pl.ds(0, D_M)],
                        out_hbm.at[rbase + j, :, pl.ds(0, D_M)],
                        osems.at[b],
                    )
                return c1

            lax.fori_loop(
                0, SLAB // NBUF, lambda t, c: block(s * SLAB + t * NBUF, c), 0
            )

            @pl.when(s + 2 < NS)
            def _restage():
                stage(s + 2)

            return carry

        lax.fori_loop(0, NS, slab_loop, 0)

        for b in range(NBUF):
            pltpu.make_async_copy(
                bufs.at[b, :, pl.ds(0, D_M)],
                out_hbm.at[rbase + RW - NBUF + b, :, pl.ds(0, D_M)],
                osems.at[b],
            ).wait()

    return body


def _wid_expr():
    return lax.axis_index("s") * 2 + lax.axis_index("c")


def kernel(x, lut):
    xi = x.astype(jnp.int32)
    lutp = jnp.pad(lut, ((0, 0), (0, PAD_W - D_M)))
    outp = _gather_call(x.shape[0], x.shape[1])(xi, lutp)
    return outp[:, :, :D_M]


# speedup vs baseline: 1.5009x; 1.0092x over previous
import functools
import jax
import jax.numpy as jnp
from jax import lax
from jax.experimental import pallas as pl
from jax.experimental.pallas import tpu as pltpu
from jax.experimental.pallas import tpu_sc as plsc

D_M = 64
PAD_W = 128
SCALE = 8.0
NW = 32
LANES = 16
NBUF = 4
AHEAD = 2
SLAB = 32


def _gather_call(R, C):
    RW = R // NW
    NS = RW // SLAB
    mesh = plsc.VectorSubcoreMesh(core_axis_name="c", subcore_axis_name="s")

    @functools.partial(
        pl.kernel,
        mesh=mesh,
        out_type=jax.ShapeDtypeStruct((R, C, PAD_W), jnp.float32),
        compiler_params=pltpu.CompilerParams(use_tc_tiling_on_sc=False),
        scratch_types=[
            pltpu.VMEM((2, SLAB, C), jnp.int32),
            pltpu.VMEM((NBUF, C, D_M), jnp.float32),
            pltpu.SemaphoreType.DMA((2,)),
            pltpu.SemaphoreType.DMA((NBUF,)),
            pltpu.SemaphoreType.DMA((NBUF,)),
        ],
    )
    def body(idx_hbm, lutp_hbm, out_hbm, islabs, bufs, isems, gsems, osems):
        rbase = _wid_expr() * RW

        def stage(s, sem_wait=False):
            args = (
                idx_hbm.at[pl.ds(rbase + s * SLAB, SLAB)],
                islabs.at[s % 2],
                isems.at[s % 2],
            )
            if sem_wait:
                pltpu.make_async_copy(*args).wait()
            else:
                pltpu.async_copy(*args)

        def idx_row(j):
            return islabs.at[(j // SLAB) % 2, j % SLAB]

        stage(0)
        stage(1)
        stage(0, sem_wait=True)
        for b in range(AHEAD):
            pltpu.async_copy(lutp_hbm.at[idx_row(b)], bufs.at[b], gsems.at[b])

        def slab_loop(s, carry):
            @pl.when(s + 1 < NS)
            def _wait_next():
                stage(s + 1, sem_wait=True)

            def block(j0, c1):
                for b in range(NBUF):
                    j = j0 + b
                    jf = j + AHEAD
                    bf = (b + AHEAD) % NBUF

                    @pl.when(jf < RW)
                    def _fire():
                        @pl.when(jf >= NBUF)
                        def _drain():
                            pltpu.make_async_copy(
                                bufs.at[bf],
                                out_hbm.at[rbase + jf - NBUF, :, pl.ds(0, D_M)],
                                osems.at[bf],
                            ).wait()

                        pltpu.async_copy(
                            lutp_hbm.at[idx_row(jf)], bufs.at[bf], gsems.at[bf]
                        )

                    pltpu.make_async_copy(
                        lutp_hbm.at[idx_row(j)], bufs.at[b], gsems.at[b]
                    ).wait()

                    def srow(t, c2):
                        for rr in range(2):
                            for q in range(D_M // LANES):
                                sl = pl.ds(q * LANES, LANES)
                                bufs[b, 2 * t + rr, sl] = (
                                    bufs[b, 2 * t + rr, sl] * SCALE
                                )
                        return c2

                    lax.fori_loop(0, C // 2, srow, 0, unroll=2)

                    pltpu.async_copy(
                        bufs.at[b],
                        out_hbm.at[rbase + j, :, pl.ds(0, D_M)],
                        osems.at[b],
                    )
                return c1

            lax.fori_loop(
                0, SLAB // NBUF, lambda t, c: block(s * SLAB + t * NBUF, c), 0
            )

            @pl.when(s + 2 < NS)
            def _restage():
                stage(s + 2)

            return carry

        lax.fori_loop(0, NS, slab_loop, 0)

        for b in range(NBUF):
            pltpu.make_async_copy(
                bufs.at[b],
                out_hbm.at[rbase + RW - NBUF + b, :, pl.ds(0, D_M)],
                osems.at[b],
            ).wait()

    return body


def _wid_expr():
    return lax.axis_index("s") * 2 + lax.axis_index("c")


def kernel(x, lut):
    xi = x.astype(jnp.int32)
    outp = _gather_call(x.shape[0], x.shape[1])(xi, lut)
    return outp[:, :, :D_M]
